# baseline (speedup 1.0000x reference)
import functools

import jax
import jax.numpy as jnp
from jax import lax
from jax.experimental import pallas as pl
from jax.experimental.pallas import tpu as pltpu
from jax.experimental.pallas import tpu_sc as plsc

EMB = 16
BATCH = 16384
BN_EPS = 1e-3
NS = 16
ROWS = BATCH // NS
GCH = 128
NCH = ROWS // GCH
LCH = ROWS // 16


def _body(uidx_hbm, bidx_hbm, uemb_hbm, ubias_hbm, bemb_hbm, bbias_hbm,
          params_hbm, out_hbm,
          uidx_v, bidx_v, urows_v, brows_v, ub_v, bb_v, params_v, acc_v,
          tot_v, out_v, shared_acc, sem):
    s = lax.axis_index("s")
    base = s * ROWS

    pltpu.sync_copy(uidx_hbm.at[pl.ds(s * NCH, NCH), :], uidx_v)
    pltpu.sync_copy(bidx_hbm.at[pl.ds(s * NCH, NCH), :], bidx_v)
    pltpu.sync_copy(params_hbm, params_v)

    cps = []
    for k in range(NCH):
        r = pl.ds(k * GCH, GCH)
        cps.append(pltpu.async_copy(uemb_hbm.at[uidx_v.at[k]], urows_v.at[r, :], sem))
        cps.append(pltpu.async_copy(bemb_hbm.at[bidx_v.at[k]], brows_v.at[r, :], sem))
        cps.append(pltpu.async_copy(ubias_hbm.at[uidx_v.at[k]], ub_v.at[r], sem))
        cps.append(pltpu.async_copy(bbias_hbm.at[bidx_v.at[k]], bb_v.at[r], sem))
    for cp in cps:
        cp.wait()

    def mac4(i, accs):
        a0, a1, a2, a3 = accs
        r = i * 4
        a0 = a0 + urows_v[r, :] * brows_v[r, :]
        a1 = a1 + urows_v[r + 1, :] * brows_v[r + 1, :]
        a2 = a2 + urows_v[r + 2, :] * brows_v[r + 2, :]
        a3 = a3 + urows_v[r + 3, :] * brows_v[r + 3, :]
        return a0, a1, a2, a3

    z = jnp.zeros((16,), jnp.float32)
    a0, a1, a2, a3 = lax.fori_loop(0, ROWS // 4, mac4, (z, z, z, z))
    acc_v[...] = (a0 + a1) + (a2 + a3)

    pltpu.sync_copy(acc_v, shared_acc.at[s])
    plsc.subcore_barrier()
    pltpu.sync_copy(shared_acc, tot_v)
    t = tot_v[0, :]
    for j in range(1, NS):
        t = t + tot_v[j, :]
    S = t[0]
    for j in range(1, 16):
        S = S + t[j]

    scale = params_v[0, :]
    shift = params_v[1, :]

    def emit(j, carry):
        c = pl.ds(j * 16, 16)
        x = (S + ub_v[c] + bb_v[c]) * scale + shift
        out_v[c] = 1.0 / (1.0 + jnp.exp(-x))
        return carry

    lax.fori_loop(0, LCH, emit, 0)
    pltpu.sync_copy(out_v, out_hbm.at[pl.ds(base, ROWS)])


@jax.jit
def _sc_call(uidx, bidx, user_emb, ubias, book_emb, bbias, params):
    mesh = plsc.VectorSubcoreMesh(core_axis_name="c", subcore_axis_name="s",
                                  num_cores=1)
    f = pl.kernel(
        _body,
        out_type=jax.ShapeDtypeStruct((BATCH,), jnp.float32),
        mesh=mesh,
        scratch_types=[
            pltpu.VMEM((NCH, GCH), jnp.int32),
            pltpu.VMEM((NCH, GCH), jnp.int32),
            pltpu.VMEM((ROWS, EMB), jnp.float32),
            pltpu.VMEM((ROWS, EMB), jnp.float32),
            pltpu.VMEM((ROWS,), jnp.float32),
            pltpu.VMEM((ROWS,), jnp.float32),
            pltpu.VMEM((2, 16), jnp.float32),
            pltpu.VMEM((16,), jnp.float32),
            pltpu.VMEM((NS, 16), jnp.float32),
            pltpu.VMEM((ROWS,), jnp.float32),
            pltpu.VMEM_SHARED((NS, 16), jnp.float32),
            pltpu.SemaphoreType.DMA,
        ],
        compiler_params=pltpu.CompilerParams(use_tc_tiling_on_sc=False),
    )
    return f(uidx, bidx, user_emb, ubias, book_emb, bbias, params)


def kernel(inputs, user_emb, user_bias, book_emb, book_bias,
           bn_gamma, bn_beta, bn_mean, bn_var):
    uidx = inputs[:, 0].reshape(NS * NCH, GCH)
    bidx = inputs[:, 1].reshape(NS * NCH, GCH)
    ubias = user_bias.reshape(-1)
    bbias = book_bias.reshape(-1)
    scale = bn_gamma * lax.rsqrt(bn_var + BN_EPS)
    shift = bn_beta - bn_mean * scale
    params = jnp.stack([jnp.broadcast_to(scale, (16,)),
                        jnp.broadcast_to(shift, (16,))])
    out = _sc_call(uidx, bidx, user_emb, ubias, book_emb, bbias, params)
    return out.reshape(BATCH, 1)
